# baseline (speedup 1.0000x reference)
import functools

import jax
import jax.numpy as jnp
from jax import lax
from jax.experimental import pallas as pl
from jax.experimental.pallas import tpu as pltpu
from jax.experimental.pallas import tpu_sc as plsc

N = 10000
E = 320000
D = 128

NPAD = 10240
CHUNK = 50
NCHUNK = E // CHUNK
NW = 32
CPW = NCHUNK // NW
ROWS_PER_TILE = NPAD // 16
RB = 6
IB = 9


def _make_sc_agg(with_cnt):
    mesh = plsc.VectorSubcoreMesh(core_axis_name="c", subcore_axis_name="s")
    out_type = [jax.ShapeDtypeStruct((2, NPAD, D), jnp.float32)]
    scratch = [
        pltpu.VMEM((IB, 2, CHUNK), jnp.int32),
        pltpu.VMEM((RB, CHUNK, D), jnp.float32),
        pltpu.VMEM((64,), jnp.float32),
        pltpu.VMEM_SHARED((NPAD, D), jnp.float32),
        pltpu.VMEM_SHARED((NPAD,), jnp.float32),
        pltpu.SemaphoreType.DMA,
        pltpu.SemaphoreType.DMA,
        pltpu.SemaphoreType.DMA,
    ]
    if with_cnt:
        out_type = out_type + [jax.ShapeDtypeStruct((2, NPAD), jnp.float32)]

    @functools.partial(pl.kernel, mesh=mesh, out_type=out_type,
                       scratch_types=scratch)
    def agg(h_hbm, e_hbm, *rest):
        if with_cnt:
            (sums_out, cnt_out,
             idx, rows, ones_v, acc, acc_cnt, sem_i, sem_g, sem_s) = rest
        else:
            (sums_out,
             idx, rows, ones_v, acc, acc_cnt, sem_i, sem_g, sem_s) = rest
        c = lax.axis_index("c")
        s = lax.axis_index("s")
        w = c * 16 + s

        zeros16 = jnp.zeros((16,), jnp.float32)
        ones16 = jnp.ones((16,), jnp.float32)

        def zrow(r, carry):
            for kk in range(8):
                rows[0, r, pl.ds(kk * 16, 16)] = zeros16
            return carry
        lax.fori_loop(0, CHUNK, zrow, 0)
        if with_cnt:
            for kk in range(4):
                ones_v[pl.ds(kk * 16, 16)] = ones16

        slab = s * ROWS_PER_TILE
        zb = 40
        for t in range(ROWS_PER_TILE // zb):
            pltpu.async_copy(rows.at[0, pl.ds(0, zb)],
                             acc.at[pl.ds(slab + t * zb, zb)], sem_s)
        if with_cnt:
            for t in range(ROWS_PER_TILE // D):
                pltpu.async_copy(rows.at[0, 0],
                                 acc_cnt.at[pl.ds(slab + t * D, D)],
                                 sem_s)
        for t in range(ROWS_PER_TILE // zb):
            pltpu.make_async_copy(
                rows.at[0, pl.ds(0, zb)], acc.at[pl.ds(slab, zb)],
                sem_s).wait()
        if with_cnt:
            for t in range(ROWS_PER_TILE // D):
                pltpu.make_async_copy(
                    rows.at[0, 0], acc_cnt.at[pl.ds(slab, D)], sem_s).wait()

        plsc.subcore_barrier()

        g0 = w * CPW

        def fetch_idx(g, slot):
            pltpu.async_copy(e_hbm.at[0, g0 + g], idx.at[slot, 0], sem_i)
            pltpu.async_copy(e_hbm.at[1, g0 + g], idx.at[slot, 1], sem_i)

        def wait_idx():
            for _ in range(2):
                pltpu.make_async_copy(
                    e_hbm.at[0, 0], idx.at[0, 0], sem_i).wait()

        pltpu.sync_copy(e_hbm.at[0, g0], idx.at[0, 0])
        pltpu.sync_copy(e_hbm.at[1, g0], idx.at[0, 1])
        for p in range(1, 6):
            fetch_idx(p, p)
        pltpu.async_copy(h_hbm.at[idx.at[0, 0]], rows.at[0], sem_g)
        wait_idx()
        pltpu.async_copy(h_hbm.at[idx.at[1, 0]], rows.at[1], sem_g)
        wait_idx()
        pltpu.async_copy(h_hbm.at[idx.at[2, 0]], rows.at[2], sem_g)

        def step(j, carry):
            u = lax.rem(j, RB)
            @pl.when(j >= 2)
            def _():
                pltpu.make_async_copy(
                    rows.at[0], acc.at[idx.at[0, 1]], sem_s).wait()
                if with_cnt:
                    pltpu.make_async_copy(
                        ones_v.at[pl.ds(0, CHUNK)], acc_cnt.at[idx.at[0, 1]], sem_s).wait()

            @pl.when(j + 3 < CPW)
            def _():
                wait_idx()
                pltpu.async_copy(
                    h_hbm.at[idx.at[lax.rem(j + 3, IB), 0]],
                    rows.at[lax.rem(j + 3, RB)], sem_g)

            pltpu.make_async_copy(
                h_hbm.at[idx.at[0, 0]], rows.at[u], sem_g).wait()
            pltpu.async_copy(rows.at[u],
                             acc.at[idx.at[lax.rem(j, IB), 1]], sem_s,
                             add=True)
            if with_cnt:
                pltpu.async_copy(ones_v.at[pl.ds(0, CHUNK)],
                                 acc_cnt.at[idx.at[lax.rem(j, IB), 1]], sem_s,
                                 add=True)

            @pl.when(j + 6 < CPW)
            def _():
                fetch_idx(j + 6, lax.rem(j + 6, IB))
            return carry
        lax.fori_loop(0, CPW, step, 0)

        for _ in range(2):
            pltpu.make_async_copy(
                rows.at[0], acc.at[idx.at[0, 1]], sem_s).wait()
            if with_cnt:
                pltpu.make_async_copy(
                    ones_v.at[pl.ds(0, CHUNK)], acc_cnt.at[idx.at[0, 1]], sem_s).wait()

        plsc.subcore_barrier()

        pltpu.sync_copy(acc.at[pl.ds(slab, ROWS_PER_TILE)],
                        sums_out.at[c, pl.ds(slab, ROWS_PER_TILE)])
        if with_cnt:
            pltpu.sync_copy(acc_cnt.at[pl.ds(slab, ROWS_PER_TILE)],
                            cnt_out.at[c, pl.ds(slab, ROWS_PER_TILE)])

    return agg


_sc_agg_cnt = _make_sc_agg(True)
_sc_agg_nocnt = _make_sc_agg(False)


def _tc_layer(h, sums_p, cnt_t, w2d, b2d):
    blk = 1000

    def body(h_ref, s_ref, c_ref, w_ref, b_ref, o_ref):
        sums = s_ref[0] + s_ref[1]
        cnt = jnp.sum(c_ref[...], axis=1, keepdims=True)
        mean = sums * (1.0 / jnp.maximum(cnt, 1.0))
        acc = jnp.dot(h_ref[...], w_ref[:D], preferred_element_type=jnp.float32)
        acc = acc + jnp.dot(mean, w_ref[D:], preferred_element_type=jnp.float32)
        o_ref[...] = jnp.maximum(acc + b_ref[...], 0.0)

    return pl.pallas_call(
        body,
        grid=(N // blk,),
        in_specs=[
            pl.BlockSpec((blk, D), lambda i: (i, 0)),
            pl.BlockSpec((2, blk, D), lambda i: (0, i, 0)),
            pl.BlockSpec((blk, 2), lambda i: (i, 0)),
            pl.BlockSpec((2 * D, D), lambda i: (0, 0)),
            pl.BlockSpec((1, D), lambda i: (0, 0)),
        ],
        out_specs=pl.BlockSpec((blk, D), lambda i: (i, 0)),
        out_shape=jax.ShapeDtypeStruct((N, D), jnp.float32),
    )(h, sums_p, cnt_t, w2d, b2d)


def kernel(h, edge_index, W1, b1, W2, b2):
    edges = edge_index.astype(jnp.int32).reshape(2, NCHUNK, CHUNK)

    sums_p, cnt_all = _sc_agg_cnt(h, edges)
    cnt_t = cnt_all.T
    h = _tc_layer(h, sums_p, cnt_t, W1, b1.reshape(1, D))
    (sums_p2,) = _sc_agg_nocnt(h, edges)
    return _tc_layer(h, sums_p2, cnt_t, W2, b2.reshape(1, D))


# speedup vs baseline: 1.0370x; 1.0370x over previous
import functools

import jax
import jax.numpy as jnp
from jax import lax
from jax.experimental import pallas as pl
from jax.experimental.pallas import tpu as pltpu
from jax.experimental.pallas import tpu_sc as plsc

N = 10000
E = 320000
D = 128

NPAD = 10240
CHUNK = 64
NCHUNK = E // CHUNK
NW = 32
CPW_LO = NCHUNK // NW
CPW_REM = NCHUNK - CPW_LO * NW
ROWS_PER_TILE = NPAD // 16
RB = 5
IB = 7


def _make_sc_agg(with_cnt):
    mesh = plsc.VectorSubcoreMesh(core_axis_name="c", subcore_axis_name="s")
    out_type = [jax.ShapeDtypeStruct((2, NPAD, D), jnp.float32)]
    scratch = [
        pltpu.VMEM((IB, 2, CHUNK), jnp.int32),
        pltpu.VMEM((RB, CHUNK, D), jnp.float32),
        pltpu.VMEM((CHUNK,), jnp.float32),
        pltpu.VMEM_SHARED((NPAD, D), jnp.float32),
        pltpu.VMEM_SHARED((NPAD,), jnp.float32),
        pltpu.SemaphoreType.DMA,
        pltpu.SemaphoreType.DMA,
        pltpu.SemaphoreType.DMA,
    ]
    if with_cnt:
        out_type = out_type + [jax.ShapeDtypeStruct((2, NPAD), jnp.float32)]

    @functools.partial(pl.kernel, mesh=mesh, out_type=out_type,
                       scratch_types=scratch)
    def agg(h_hbm, e_hbm, *rest):
        if with_cnt:
            (sums_out, cnt_out,
             idx, rows, ones_v, acc, acc_cnt, sem_i, sem_g, sem_s) = rest
        else:
            (sums_out,
             idx, rows, ones_v, acc, acc_cnt, sem_i, sem_g, sem_s) = rest
        c = lax.axis_index("c")
        s = lax.axis_index("s")
        w = c * 16 + s
        ncw = CPW_LO + jnp.where(w < CPW_REM, 1, 0)
        e0 = (w * CPW_LO + jnp.minimum(w, CPW_REM)) * CHUNK

        zeros16 = jnp.zeros((16,), jnp.float32)
        ones16 = jnp.ones((16,), jnp.float32)

        def zrow(r, carry):
            for kk in range(8):
                rows[0, r, pl.ds(kk * 16, 16)] = zeros16
            return carry
        lax.fori_loop(0, CHUNK, zrow, 0)
        if with_cnt:
            for kk in range(CHUNK // 16):
                ones_v[pl.ds(kk * 16, 16)] = ones16

        slab = s * ROWS_PER_TILE
        for t in range(ROWS_PER_TILE // CHUNK):
            pltpu.async_copy(rows.at[0], acc.at[pl.ds(slab + t * CHUNK, CHUNK)],
                             sem_s)
        if with_cnt:
            for t in range(ROWS_PER_TILE // D):
                pltpu.async_copy(rows.at[0, 0],
                                 acc_cnt.at[pl.ds(slab + t * D, D)],
                                 sem_s)
        for t in range(ROWS_PER_TILE // CHUNK):
            pltpu.make_async_copy(
                rows.at[0], acc.at[pl.ds(slab, CHUNK)], sem_s).wait()
        if with_cnt:
            for t in range(ROWS_PER_TILE // D):
                pltpu.make_async_copy(
                    rows.at[0, 0], acc_cnt.at[pl.ds(slab, D)], sem_s).wait()

        plsc.subcore_barrier()

        g0 = e0 // CHUNK

        def fetch_idx(g, slot):
            pltpu.async_copy(e_hbm.at[0, g0 + g], idx.at[slot, 0], sem_i)
            pltpu.async_copy(e_hbm.at[1, g0 + g], idx.at[slot, 1], sem_i)

        def wait_idx():
            for _ in range(2):
                pltpu.make_async_copy(
                    e_hbm.at[0, 0], idx.at[0, 0], sem_i).wait()

        pltpu.sync_copy(e_hbm.at[0, g0], idx.at[0, 0])
        pltpu.sync_copy(e_hbm.at[1, g0], idx.at[0, 1])
        for p in range(1, 5):
            fetch_idx(p, p)
        pltpu.async_copy(h_hbm.at[idx.at[0, 0]], rows.at[0], sem_g)
        wait_idx()
        pltpu.async_copy(h_hbm.at[idx.at[1, 0]], rows.at[1], sem_g)

        def step(j, carry):
            u = lax.rem(j, RB)
            @pl.when(j >= 2)
            def _():
                pltpu.make_async_copy(
                    rows.at[0], acc.at[idx.at[0, 1]], sem_s).wait()
                if with_cnt:
                    pltpu.make_async_copy(
                        ones_v, acc_cnt.at[idx.at[0, 1]], sem_s).wait()

            @pl.when(j + 2 < ncw)
            def _():
                wait_idx()
                pltpu.async_copy(
                    h_hbm.at[idx.at[lax.rem(j + 2, IB), 0]],
                    rows.at[lax.rem(j + 2, RB)], sem_g)

            pltpu.make_async_copy(
                h_hbm.at[idx.at[0, 0]], rows.at[u], sem_g).wait()
            pltpu.async_copy(rows.at[u],
                             acc.at[idx.at[lax.rem(j, IB), 1]], sem_s,
                             add=True)
            if with_cnt:
                pltpu.async_copy(ones_v,
                                 acc_cnt.at[idx.at[lax.rem(j, IB), 1]], sem_s,
                                 add=True)

            @pl.when(j + 5 < ncw)
            def _():
                fetch_idx(j + 5, lax.rem(j + 5, IB))
            return carry
        lax.fori_loop(0, ncw, step, 0)

        for _ in range(2):
            pltpu.make_async_copy(
                rows.at[0], acc.at[idx.at[0, 1]], sem_s).wait()
            if with_cnt:
                pltpu.make_async_copy(
                    ones_v, acc_cnt.at[idx.at[0, 1]], sem_s).wait()

        plsc.subcore_barrier()

        pltpu.sync_copy(acc.at[pl.ds(slab, ROWS_PER_TILE)],
                        sums_out.at[c, pl.ds(slab, ROWS_PER_TILE)])
        if with_cnt:
            pltpu.sync_copy(acc_cnt.at[pl.ds(slab, ROWS_PER_TILE)],
                            cnt_out.at[c, pl.ds(slab, ROWS_PER_TILE)])

    return agg


_sc_agg_cnt = _make_sc_agg(True)
_sc_agg_nocnt = _make_sc_agg(False)


def _tc_layer(h, sums_p, cnt_t, w2d, b2d):
    blk = 1000

    def body(h_ref, s_ref, c_ref, w_ref, b_ref, o_ref):
        sums = s_ref[0] + s_ref[1]
        cnt = jnp.sum(c_ref[...], axis=1, keepdims=True)
        mean = sums * (1.0 / jnp.maximum(cnt, 1.0))
        acc = jnp.dot(h_ref[...], w_ref[:D], preferred_element_type=jnp.float32)
        acc = acc + jnp.dot(mean, w_ref[D:], preferred_element_type=jnp.float32)
        o_ref[...] = jnp.maximum(acc + b_ref[...], 0.0)

    return pl.pallas_call(
        body,
        grid=(N // blk,),
        in_specs=[
            pl.BlockSpec((blk, D), lambda i: (i, 0)),
            pl.BlockSpec((2, blk, D), lambda i: (0, i, 0)),
            pl.BlockSpec((blk, 2), lambda i: (i, 0)),
            pl.BlockSpec((2 * D, D), lambda i: (0, 0)),
            pl.BlockSpec((1, D), lambda i: (0, 0)),
        ],
        out_specs=pl.BlockSpec((blk, D), lambda i: (i, 0)),
        out_shape=jax.ShapeDtypeStruct((N, D), jnp.float32),
    )(h, sums_p, cnt_t, w2d, b2d)


def kernel(h, edge_index, W1, b1, W2, b2):
    edges = edge_index.astype(jnp.int32).reshape(2, NCHUNK, CHUNK)

    sums_p, cnt_all = _sc_agg_cnt(h, edges)
    cnt_t = cnt_all.T
    h = _tc_layer(h, sums_p, cnt_t, W1, b1.reshape(1, D))
    (sums_p2,) = _sc_agg_nocnt(h, edges)
    return _tc_layer(h, sums_p2, cnt_t, W2, b2.reshape(1, D))
